# baseline (speedup 1.0000x reference)
import functools

import jax
import jax.numpy as jnp
from jax import lax
from jax.experimental import pallas as pl
from jax.experimental.pallas import tpu as pltpu
from jax.experimental.pallas import tpu_sc as plsc

LENGTH = 577
S = 24
NU = 64
NSEG = NU // 16
TROWS = 30
MAXREL = 14

_info = plsc.get_sparse_core_info()
NC = _info.num_cores
NS = _info.num_subcores
NW = NC * NS


def _clip15(x):
    return jnp.minimum(jnp.maximum(x, -MAXREL), MAXREL) + 15


def _sc_body(tv_hbm, th_hbm, out_hbm, tv_v, th_v, bb_v, buf_v, sem0, sem1):
    w = lax.axis_index("s") * NC + lax.axis_index("c")
    sems = (sem0, sem1)

    pltpu.sync_copy(tv_hbm, tv_v)
    pltpu.sync_copy(th_hbm, th_v)
    c0 = [tv_v[0, pl.ds(16 * l, 16)] + th_v[0, pl.ds(16 * l, 16)]
          for l in range(NSEG)]

    def fill_row(i, b):
        bref = buf_v.at[b]
        for l in range(NSEG):
            bref[0, pl.ds(16 * l, 16)] = c0[l]

        @pl.when(i == 0)
        def _():
            def body0(j, carry):
                for l in range(NSEG):
                    bref[j, pl.ds(16 * l, 16)] = c0[l]
                return carry
            lax.fori_loop(1, LENGTH, body0, 0)

        @pl.when(i > 0)
        def _():
            r = i - 1
            rv = r // S
            rh = lax.rem(r, S)
            for ch in range(S):
                h = _clip15(ch - rh)
                for l in range(NSEG):
                    bb_v[ch, pl.ds(16 * l, 16)] = th_v[h, pl.ds(16 * l, 16)]

            def cvbody(cv, carry):
                a = _clip15(cv - rv)
                va = [tv_v[a, pl.ds(16 * l, 16)] for l in range(NSEG)]
                base = 1 + cv * S
                for ch in range(S):
                    for l in range(NSEG):
                        bref[base + ch, pl.ds(16 * l, 16)] = (
                            va[l] + bb_v[ch, pl.ds(16 * l, 16)])
                return carry
            lax.fori_loop(0, S, cvbody, 0)

    def outer(t, carry):
        for b in range(2):
            k = 2 * t + b
            i = NW * k + w

            @pl.when(k >= 2)
            def _():
                pltpu.make_async_copy(buf_v.at[b], out_hbm.at[0],
                                      sems[b]).wait()

            @pl.when(i < LENGTH)
            def _():
                fill_row(i, b)
                pltpu.async_copy(buf_v.at[b], out_hbm.at[i], sems[b])
        return carry
    lax.fori_loop(0, 10, outer, 0)

    @pl.when(w == 0)
    def _():
        pltpu.make_async_copy(buf_v.at[0], out_hbm.at[0], sems[0]).wait()


@functools.partial(
    pl.kernel,
    mesh=plsc.VectorSubcoreMesh(core_axis_name="c", subcore_axis_name="s"),
    out_type=jax.ShapeDtypeStruct((LENGTH, LENGTH, NU), jnp.float32),
    scratch_types=[
        pltpu.VMEM((TROWS, NU), jnp.float32),
        pltpu.VMEM((TROWS, NU), jnp.float32),
        pltpu.VMEM((S, NU), jnp.float32),
        pltpu.VMEM((2, LENGTH, NU), jnp.float32),
        pltpu.SemaphoreType.DMA,
        pltpu.SemaphoreType.DMA,
    ],
    compiler_params=pltpu.CompilerParams(use_tc_tiling_on_sc=False),
)
def _sc_rel_pos(tv_hbm, th_hbm, out_hbm, tv_v, th_v, bb_v, buf_v, sem0, sem1):
    _sc_body(tv_hbm, th_hbm, out_hbm, tv_v, th_v, bb_v, buf_v, sem0, sem1)


def kernel(table_v, table_h, length_q, length_k):
    del length_q, length_k
    return _sc_rel_pos(table_v, table_h)


# speedup vs baseline: 1.8199x; 1.8199x over previous
import functools

import jax
import jax.numpy as jnp
from jax import lax
from jax.experimental import pallas as pl
from jax.experimental.pallas import tpu as pltpu
from jax.experimental.pallas import tpu_sc as plsc

LENGTH = 577
S = 24
NU = 64
NSEG = NU // 16
TROWS = 30
MAXREL = 14
TP = 65
NCHUNK = 37

_info = plsc.get_sparse_core_info()
NC = _info.num_cores
NS = _info.num_subcores
NW = NC * NS


def _clip15(x):
    return jnp.minimum(jnp.maximum(x, -MAXREL), MAXREL) + 15


def _sc_body(tv_hbm, th_hbm, out_hbm, tv_raw, th_raw, tvp_v, thp_v, buf_v,
             sem0, sem1):
    w = lax.axis_index("s") * NC + lax.axis_index("c")
    sems = (sem0, sem1)

    pltpu.sync_copy(tv_hbm, tv_raw)
    pltpu.sync_copy(th_hbm, th_raw)
    for rr in range(TROWS):
        for l in range(NSEG):
            tvp_v[pl.ds(TP * rr + 16 * l, 16)] = tv_raw[rr, pl.ds(16 * l, 16)]
            thp_v[pl.ds(TP * rr + 16 * l, 16)] = th_raw[rr, pl.ds(16 * l, 16)]

    iota = lax.iota(jnp.int32, 16)

    def fill_row(i, b):
        bref = buf_v.at[b]
        r = i - 1
        rv = r // S
        rh = r % S
        row_valid = i >= 1

        def mbody(m, carry):
            joff = jnp.minimum(16 * m, LENGTH - 16)
            jv = iota + joff
            jm1 = jv - 1
            cv0 = (joff - 1) // S
            bnd = (cv0 + 1) * S + 1 - joff
            cvj = cv0 + jnp.where(iota >= bnd, 1, 0)
            chj = jm1 - S * cvj
            avc = _clip15(cvj - rv)
            bvc = _clip15(chj - rh)
            valid = jnp.logical_and(jv >= 1, row_valid)
            iva = jnp.where(valid, avc, 0) * TP
            ihb = jnp.where(valid, bvc, 0) * TP
            for d in range(NU):
                ga = plsc.load_gather(tvp_v, [iva + d])
                gb = plsc.load_gather(thp_v, [ihb + d])
                bref[d, pl.ds(joff, 16)] = ga + gb
            return carry
        lax.fori_loop(0, NCHUNK, mbody, 0)

    def outer(t, carry):
        for b in range(2):
            k = 2 * t + b
            i = NW * k + w

            @pl.when(k >= 2)
            def _():
                pltpu.make_async_copy(buf_v.at[b],
                                      out_hbm.at[0], sems[b]).wait()

            @pl.when(i < LENGTH)
            def _():
                fill_row(i, b)
                pltpu.async_copy(buf_v.at[b],
                                 out_hbm.at[i], sems[b])
        return carry
    lax.fori_loop(0, 10, outer, 0)

    @pl.when(w == 0)
    def _():
        pltpu.make_async_copy(buf_v.at[0],
                              out_hbm.at[0], sems[0]).wait()


@functools.partial(
    pl.kernel,
    mesh=plsc.VectorSubcoreMesh(core_axis_name="c", subcore_axis_name="s"),
    out_type=jax.ShapeDtypeStruct((LENGTH, NU, LENGTH), jnp.float32),
    scratch_types=[
        pltpu.VMEM((TROWS, NU), jnp.float32),
        pltpu.VMEM((TROWS, NU), jnp.float32),
        pltpu.VMEM((TROWS * TP,), jnp.float32),
        pltpu.VMEM((TROWS * TP,), jnp.float32),
        pltpu.VMEM((2, NU, LENGTH), jnp.float32),
        pltpu.SemaphoreType.DMA,
        pltpu.SemaphoreType.DMA,
    ],
    compiler_params=pltpu.CompilerParams(needs_layout_passes=False),
)
def _sc_rel_pos(tv_hbm, th_hbm, out_hbm, tv_raw, th_raw, tvp_v, thp_v, buf_v,
                sem0, sem1):
    _sc_body(tv_hbm, th_hbm, out_hbm, tv_raw, th_raw, tvp_v, thp_v, buf_v,
             sem0, sem1)


def kernel(table_v, table_h, length_q, length_k):
    del length_q, length_k
    out = _sc_rel_pos(table_v, table_h)
    return jnp.transpose(out, (0, 2, 1))


# speedup vs baseline: 5.4708x; 3.0062x over previous
import functools

import jax
import jax.numpy as jnp
from jax import lax
from jax.experimental import pallas as pl
from jax.experimental.pallas import tpu as pltpu
from jax.experimental.pallas import tpu_sc as plsc

LENGTH = 577
S = 24
NU = 64
NSEG = NU // 16
TROWS = 30
MAXREL = 14
TP = 65
NCHUNK = 37

_info = plsc.get_sparse_core_info()
NC = _info.num_cores
NS = _info.num_subcores
NW = NC * NS


def _clip15(x):
    return jnp.minimum(jnp.maximum(x, -MAXREL), MAXREL) + 15


def _sc_body(tv_hbm, th_hbm, out_hbm, tv_raw, th_raw, tvp_v, thp_v, buf_v,
             sem0, sem1):
    w = lax.axis_index("s") * NC + lax.axis_index("c")
    sems = (sem0, sem1)

    pltpu.sync_copy(tv_hbm, tv_raw)
    pltpu.sync_copy(th_hbm, th_raw)
    for rr in range(TROWS):
        for l in range(NSEG):
            tvp_v[pl.ds(TP * rr + 16 * l, 16)] = tv_raw[rr, pl.ds(16 * l, 16)]
            thp_v[pl.ds(TP * rr + 16 * l, 16)] = th_raw[rr, pl.ds(16 * l, 16)]

    iota = lax.iota(jnp.int32, 16)

    def fill_row(i, b):
        bref = buf_v.at[b]
        r = i - 1
        rv = r // S
        rh = r % S
        row_valid = i >= 1

        @plsc.parallel_loop(0, NCHUNK)
        def mbody(m):
            joff = jnp.minimum(16 * m, LENGTH - 16)
            jv = iota + joff
            jm1 = jv - 1
            cv0 = (joff - 1) // S
            bnd = (cv0 + 1) * S + 1 - joff
            cvj = cv0 + jnp.where(iota >= bnd, 1, 0)
            chj = jm1 - S * cvj
            avc = _clip15(cvj - rv)
            bvc = _clip15(chj - rh)
            valid = jnp.logical_and(jv >= 1, row_valid)
            iva = jnp.where(valid, avc, 0) * TP
            ihb = jnp.where(valid, bvc, 0) * TP
            for d in range(NU):
                ga = plsc.load_gather(tvp_v, [iva + d])
                gb = plsc.load_gather(thp_v, [ihb + d])
                bref[d, pl.ds(joff, 16)] = ga + gb

    def outer(t, carry):
        for b in range(2):
            k = 2 * t + b
            i = NW * k + w

            @pl.when(k >= 2)
            def _():
                pltpu.make_async_copy(buf_v.at[b],
                                      out_hbm.at[0], sems[b]).wait()

            @pl.when(i < LENGTH)
            def _():
                fill_row(i, b)
                pltpu.async_copy(buf_v.at[b],
                                 out_hbm.at[i], sems[b])
        return carry
    lax.fori_loop(0, 10, outer, 0)

    @pl.when(w == 0)
    def _():
        pltpu.make_async_copy(buf_v.at[0],
                              out_hbm.at[0], sems[0]).wait()


@functools.partial(
    pl.kernel,
    mesh=plsc.VectorSubcoreMesh(core_axis_name="c", subcore_axis_name="s"),
    out_type=jax.ShapeDtypeStruct((LENGTH, NU, LENGTH), jnp.float32),
    scratch_types=[
        pltpu.VMEM((TROWS, NU), jnp.float32),
        pltpu.VMEM((TROWS, NU), jnp.float32),
        pltpu.VMEM((TROWS * TP,), jnp.float32),
        pltpu.VMEM((TROWS * TP,), jnp.float32),
        pltpu.VMEM((2, NU, LENGTH), jnp.float32),
        pltpu.SemaphoreType.DMA,
        pltpu.SemaphoreType.DMA,
    ],
    compiler_params=pltpu.CompilerParams(needs_layout_passes=False),
)
def _sc_rel_pos(tv_hbm, th_hbm, out_hbm, tv_raw, th_raw, tvp_v, thp_v, buf_v,
                sem0, sem1):
    _sc_body(tv_hbm, th_hbm, out_hbm, tv_raw, th_raw, tvp_v, thp_v, buf_v,
             sem0, sem1)


def kernel(table_v, table_h, length_q, length_k):
    del length_q, length_k
    out = _sc_rel_pos(table_v, table_h)
    return jnp.transpose(out, (0, 2, 1))
